# baseline (speedup 1.0000x reference)
import functools

import jax
import jax.numpy as jnp
from jax import lax
from jax.experimental import pallas as pl
from jax.experimental.pallas import tpu as pltpu
from jax.experimental.pallas import tpu_sc as plsc

N = 65536
D_FEAT = 128
C = 20
K = 8192
H = 256

NSUB = 32
GROWS = K // NSUB
SROWS = (N // 128) // NSUB
PAD = 2048

_SC_MESH = functools.partial(
    plsc.VectorSubcoreMesh, core_axis_name="c", subcore_axis_name="s")


_SCORE_R = 4096


def _score_body(l_ref, keys_ref):
    l = l_ref[...]
    lmax = jnp.max(l, axis=-1, keepdims=True)
    e = jnp.exp(l - lmax)
    p = e / jnp.sum(e, axis=-1, keepdims=True)
    s = -jnp.sum(p * jnp.log(p + 1e-9), axis=-1)
    u = lax.bitcast_convert_type(s, jnp.uint32)
    mask = jnp.where(u >> 31 == 1, jnp.uint32(0xFFFFFFFF), jnp.uint32(0x80000000))
    keys_ref[...] = (u ^ mask).reshape(_SCORE_R // 128, 128)


def _scores(logits):
    return pl.pallas_call(
        _score_body,
        grid=(N // _SCORE_R,),
        in_specs=[pl.BlockSpec((_SCORE_R, C), lambda i: (i, 0))],
        out_specs=pl.BlockSpec((_SCORE_R // 128, 128), lambda i: (i, 0)),
        out_shape=jax.ShapeDtypeStruct((N // 128, 128), jnp.uint32),
    )(logits)


def _slots_body(keys_ref, slots_ref, vals_ref):
    keys = keys_ref[...]

    def step(i, prefix):
        cand = prefix | (jnp.uint32(1) << jnp.uint32(31 - i))
        cnt = jnp.sum((keys >= cand).astype(jnp.int32))
        return jnp.where(cnt >= K, cand, prefix)

    t = lax.fori_loop(0, 32, step, jnp.uint32(0))
    cnt_gt = jnp.sum((keys > t).astype(jnp.int32))
    m = (K - cnt_gt).astype(jnp.float32)

    gt = keys > t
    eq = keys == t
    eq_f = eq.astype(jnp.float32)

    nrow = keys.shape[0]
    tri_l = (lax.broadcasted_iota(jnp.int32, (128, 128), 0)
             <= lax.broadcasted_iota(jnp.int32, (128, 128), 1)).astype(jnp.float32)
    tri_r = (lax.broadcasted_iota(jnp.int32, (nrow, nrow), 1)
             < lax.broadcasted_iota(jnp.int32, (nrow, nrow), 0)).astype(jnp.float32)

    ceq = jnp.dot(eq_f, tri_l, preferred_element_type=jnp.float32)
    eq_off = jnp.dot(tri_r, ceq[:, 127:128], preferred_element_type=jnp.float32)
    tie_rank = ceq + eq_off
    sel = gt | (eq & (tie_rank <= m))

    sel_f = sel.astype(jnp.float32)
    csel = jnp.dot(sel_f, tri_l, preferred_element_type=jnp.float32)
    sel_off = jnp.dot(tri_r, csel[:, 127:128], preferred_element_type=jnp.float32)
    pos = (csel + sel_off).astype(jnp.int32) - 1

    flat = (128 * lax.broadcasted_iota(jnp.int32, keys.shape, 0)
            + lax.broadcasted_iota(jnp.int32, keys.shape, 1))
    slots_ref[...] = jnp.where(sel, pos, K + ((flat * 7) & (PAD - 1)))
    vals_ref[...] = flat


def _slots(keys):
    return pl.pallas_call(
        _slots_body,
        in_specs=[pl.BlockSpec(memory_space=pltpu.VMEM)],
        out_specs=[
            pl.BlockSpec(memory_space=pltpu.VMEM),
            pl.BlockSpec(memory_space=pltpu.VMEM),
        ],
        out_shape=[
            jax.ShapeDtypeStruct((N // 128, 128), jnp.int32),
            jax.ShapeDtypeStruct((N // 128, 128), jnp.int32),
        ],
    )(keys)


def _compact(slots, vals):
    @functools.partial(
        pl.kernel,
        mesh=_SC_MESH(),
        out_type=jax.ShapeDtypeStruct((K + PAD,), jnp.int32),
        scratch_types=[
            pltpu.VMEM((SROWS, 128), jnp.int32),
            pltpu.VMEM((SROWS, 128), jnp.int32),
            pltpu.SemaphoreType.DMA,
        ],
    )
    def body(slots_hbm, vals_hbm, out_hbm, slot_v, val_v, sem):
        wid = lax.axis_index("s") * 2 + lax.axis_index("c")
        pltpu.sync_copy(slots_hbm.at[pl.ds(wid * SROWS, SROWS)], slot_v)
        pltpu.sync_copy(vals_hbm.at[pl.ds(wid * SROWS, SROWS)], val_v)
        for r in range(SROWS):
            pltpu.async_copy(val_v.at[r], out_hbm.at[slot_v.at[r]], sem)
        for r in range(SROWS):
            pltpu.make_async_copy(val_v.at[r], out_hbm.at[slot_v.at[r]], sem).wait()

    return body(slots, vals)


def _gather(feat, label, idx):
    @functools.partial(
        pl.kernel,
        mesh=_SC_MESH(),
        out_type=[
            jax.ShapeDtypeStruct((K, D_FEAT), jnp.float32),
            jax.ShapeDtypeStruct((K,), jnp.int32),
        ],
        scratch_types=[
            pltpu.VMEM((GROWS // 128, 128), jnp.int32),
            pltpu.VMEM((GROWS, D_FEAT), jnp.float32),
            pltpu.VMEM((GROWS,), jnp.int32),
            pltpu.SemaphoreType.DMA,
            pltpu.SemaphoreType.DMA,
        ],
    )
    def body(feat_hbm, label_hbm, idx_hbm,
             fs_hbm, lm_hbm, idx_v, fs_v, lm_v, sem1, sem2):
        wid = lax.axis_index("s") * 2 + lax.axis_index("c")
        base = wid * GROWS
        pltpu.sync_copy(idx_hbm.at[pl.ds(wid * (GROWS // 128), GROWS // 128)], idx_v)
        for r in range(GROWS // 128):
            pltpu.async_copy(feat_hbm.at[idx_v.at[r]],
                             fs_v.at[pl.ds(r * 128, 128)], sem1)
            pltpu.async_copy(label_hbm.at[idx_v.at[r]],
                             lm_v.at[pl.ds(r * 128, 128)], sem2)
        for r in range(GROWS // 128):
            pltpu.make_async_copy(feat_hbm.at[idx_v.at[r]],
                                  fs_v.at[pl.ds(r * 128, 128)], sem1).wait()
            pltpu.make_async_copy(label_hbm.at[idx_v.at[r]],
                                  lm_v.at[pl.ds(r * 128, 128)], sem2).wait()
        pltpu.sync_copy(fs_v, fs_hbm.at[pl.ds(base, GROWS)])
        pltpu.sync_copy(lm_v, lm_hbm.at[pl.ds(base, GROWS)])

    return body(feat, label, idx)


_MLP_R = 1024


def _mlp_body(fs_ref, yi_ref, w1f_ref, w1l_ref, b1_ref, w2_ref, b2_ref,
              w3_ref, b3_ref, wc_ref, bc_ref, alpha_ref, yo_ref, ye_ref):
    x = fs_ref[...]
    yi = yi_ref[...]
    h = (jnp.dot(x, w1f_ref[...], preferred_element_type=jnp.float32)
         + jnp.dot(yi, w1l_ref[...], preferred_element_type=jnp.float32)
         + b1_ref[...])
    h = jnp.maximum(h, 0.0)
    h = jnp.maximum(jnp.dot(h, w2_ref[...], preferred_element_type=jnp.float32)
                    + b2_ref[...], 0.0)
    h = h + jnp.maximum(jnp.dot(h, w3_ref[...], preferred_element_type=jnp.float32)
                        + b3_ref[...], 0.0)
    yo = jnp.dot(h, wc_ref[...], preferred_element_type=jnp.float32) + bc_ref[...]
    alpha = alpha_ref[0, 0]
    yo_ref[...] = yo
    ye_ref[...] = alpha * yi + (1.0 - alpha) * yo


def _mlp(fs, yi, W1, b1, W2, b2, W3, b3, Wc, bc, alpha):
    w1f = W1[:D_FEAT]
    w1l = W1[D_FEAT:]
    return pl.pallas_call(
        _mlp_body,
        grid=(K // _MLP_R,),
        in_specs=[
            pl.BlockSpec((_MLP_R, D_FEAT), lambda i: (i, 0)),
            pl.BlockSpec((_MLP_R, C), lambda i: (i, 0)),
            pl.BlockSpec((D_FEAT, H), lambda i: (0, 0)),
            pl.BlockSpec((C, H), lambda i: (0, 0)),
            pl.BlockSpec((1, H), lambda i: (0, 0)),
            pl.BlockSpec((H, H), lambda i: (0, 0)),
            pl.BlockSpec((1, H), lambda i: (0, 0)),
            pl.BlockSpec((H, H), lambda i: (0, 0)),
            pl.BlockSpec((1, H), lambda i: (0, 0)),
            pl.BlockSpec((H, C), lambda i: (0, 0)),
            pl.BlockSpec((1, C), lambda i: (0, 0)),
            pl.BlockSpec(memory_space=pltpu.SMEM),
        ],
        out_specs=[
            pl.BlockSpec((_MLP_R, C), lambda i: (i, 0)),
            pl.BlockSpec((_MLP_R, C), lambda i: (i, 0)),
        ],
        out_shape=[
            jax.ShapeDtypeStruct((K, C), jnp.float32),
            jax.ShapeDtypeStruct((K, C), jnp.float32),
        ],
    )(fs, yi, w1f, w1l, b1.reshape(1, H), W2, b2.reshape(1, H), W3,
      b3.reshape(1, H), Wc, bc.reshape(1, C), alpha.reshape(1, 1))


def kernel(feat, logits, label, W1, b1, W2, b2, W3, b3, Wc, bc, g):
    keys2d = _scores(logits)
    slots, vals = _slots(keys2d)
    idx_padded = _compact(slots, vals)
    idx = idx_padded[:K]

    fs, label_mask = _gather(feat, label, idx.reshape(K // 128, 128))
    yi = jnp.take(logits, idx, axis=0)
    alpha = jax.nn.sigmoid(g)
    yo, ye = _mlp(fs, yi, W1, b1, W2, b2, W3, b3, Wc, bc, alpha)

    yi_full = logits
    yo_full = logits.at[idx].set(yo, unique_indices=True)
    ye_full = logits.at[idx].set(ye, unique_indices=True)
    return (yi, yo, ye, yi_full, yo_full, ye_full, label, label_mask)


# speedup vs baseline: 2.0249x; 2.0249x over previous
import functools

import jax
import jax.numpy as jnp
from jax import lax
from jax.experimental import pallas as pl
from jax.experimental.pallas import tpu as pltpu
from jax.experimental.pallas import tpu_sc as plsc

N = 65536
D_FEAT = 128
C = 20
K = 8192
H = 256

NSUB = 32
GROWS = K // NSUB
SROWS = (N // 128) // NSUB
PAD = N - K

_SC_MESH = functools.partial(
    plsc.VectorSubcoreMesh, core_axis_name="c", subcore_axis_name="s")


_SCORE_R = 4096


def _score_body(l_ref, keys_ref):
    l = l_ref[...]
    lmax = jnp.max(l, axis=-1, keepdims=True)
    e = jnp.exp(l - lmax)
    p = e / jnp.sum(e, axis=-1, keepdims=True)
    s = -jnp.sum(p * jnp.log(p + 1e-9), axis=-1)
    u = lax.bitcast_convert_type(s, jnp.uint32)
    mask = jnp.where(u >> 31 == 1, jnp.uint32(0xFFFFFFFF), jnp.uint32(0x80000000))
    keys_ref[...] = (u ^ mask).reshape(_SCORE_R // 128, 128)


def _scores(logits):
    return pl.pallas_call(
        _score_body,
        grid=(N // _SCORE_R,),
        in_specs=[pl.BlockSpec((_SCORE_R, C), lambda i: (i, 0))],
        out_specs=pl.BlockSpec((_SCORE_R // 128, 128), lambda i: (i, 0)),
        out_shape=jax.ShapeDtypeStruct((N // 128, 128), jnp.uint32),
    )(logits)


def _slots_body(keys_ref, slots_ref, vals_ref):
    keys = keys_ref[...]

    def step(i, prefix):
        cand = prefix | (jnp.uint32(1) << jnp.uint32(31 - i))
        cnt = jnp.sum((keys >= cand).astype(jnp.int32))
        return jnp.where(cnt >= K, cand, prefix)

    t = lax.fori_loop(0, 32, step, jnp.uint32(0))
    cnt_gt = jnp.sum((keys > t).astype(jnp.int32))
    m = (K - cnt_gt).astype(jnp.float32)

    gt = keys > t
    eq = keys == t
    eq_f = eq.astype(jnp.float32)

    nrow = keys.shape[0]
    tri_l = (lax.broadcasted_iota(jnp.int32, (128, 128), 0)
             <= lax.broadcasted_iota(jnp.int32, (128, 128), 1)).astype(jnp.float32)
    tri_r = (lax.broadcasted_iota(jnp.int32, (nrow, nrow), 1)
             < lax.broadcasted_iota(jnp.int32, (nrow, nrow), 0)).astype(jnp.float32)

    ceq = jnp.dot(eq_f, tri_l, preferred_element_type=jnp.float32)
    eq_off = jnp.dot(tri_r, ceq[:, 127:128], preferred_element_type=jnp.float32)
    tie_rank = ceq + eq_off
    sel = gt | (eq & (tie_rank <= m))

    sel_f = sel.astype(jnp.float32)
    csel = jnp.dot(sel_f, tri_l, preferred_element_type=jnp.float32)
    sel_off = jnp.dot(tri_r, csel[:, 127:128], preferred_element_type=jnp.float32)
    pos = (csel + sel_off).astype(jnp.int32) - 1

    flat = (128 * lax.broadcasted_iota(jnp.int32, keys.shape, 0)
            + lax.broadcasted_iota(jnp.int32, keys.shape, 1))
    slots_ref[...] = jnp.where(sel, pos, K + flat - (pos + 1))
    vals_ref[...] = flat


def _slots(keys):
    return pl.pallas_call(
        _slots_body,
        in_specs=[pl.BlockSpec(memory_space=pltpu.VMEM)],
        out_specs=[
            pl.BlockSpec(memory_space=pltpu.VMEM),
            pl.BlockSpec(memory_space=pltpu.VMEM),
        ],
        out_shape=[
            jax.ShapeDtypeStruct((N // 128, 128), jnp.int32),
            jax.ShapeDtypeStruct((N // 128, 128), jnp.int32),
        ],
    )(keys)


def _compact(slots, vals):
    @functools.partial(
        pl.kernel,
        mesh=_SC_MESH(),
        out_type=jax.ShapeDtypeStruct((K + PAD,), jnp.int32),
        scratch_types=[
            pltpu.VMEM((SROWS, 128), jnp.int32),
            pltpu.VMEM((SROWS, 128), jnp.int32),
            pltpu.SemaphoreType.DMA,
        ],
    )
    def body(slots_hbm, vals_hbm, out_hbm, slot_v, val_v, sem):
        wid = lax.axis_index("s") * 2 + lax.axis_index("c")
        pltpu.sync_copy(slots_hbm.at[pl.ds(wid * SROWS, SROWS)], slot_v)
        pltpu.sync_copy(vals_hbm.at[pl.ds(wid * SROWS, SROWS)], val_v)
        for r in range(SROWS):
            pltpu.async_copy(val_v.at[r], out_hbm.at[slot_v.at[r]], sem)
        for r in range(SROWS):
            pltpu.make_async_copy(val_v.at[r], out_hbm.at[slot_v.at[r]], sem).wait()

    return body(slots, vals)


def _gather(feat, label, idx):
    @functools.partial(
        pl.kernel,
        mesh=_SC_MESH(),
        out_type=[
            jax.ShapeDtypeStruct((K, D_FEAT), jnp.float32),
            jax.ShapeDtypeStruct((K,), jnp.int32),
        ],
        scratch_types=[
            pltpu.VMEM((GROWS // 128, 128), jnp.int32),
            pltpu.VMEM((GROWS, D_FEAT), jnp.float32),
            pltpu.VMEM((GROWS,), jnp.int32),
            pltpu.SemaphoreType.DMA,
            pltpu.SemaphoreType.DMA,
        ],
    )
    def body(feat_hbm, label_hbm, idx_hbm,
             fs_hbm, lm_hbm, idx_v, fs_v, lm_v, sem1, sem2):
        wid = lax.axis_index("s") * 2 + lax.axis_index("c")
        base = wid * GROWS
        pltpu.sync_copy(idx_hbm.at[pl.ds(wid * (GROWS // 128), GROWS // 128)], idx_v)
        for r in range(GROWS // 128):
            pltpu.async_copy(feat_hbm.at[idx_v.at[r]],
                             fs_v.at[pl.ds(r * 128, 128)], sem1)
            pltpu.async_copy(label_hbm.at[idx_v.at[r]],
                             lm_v.at[pl.ds(r * 128, 128)], sem2)
        for r in range(GROWS // 128):
            pltpu.make_async_copy(feat_hbm.at[idx_v.at[r]],
                                  fs_v.at[pl.ds(r * 128, 128)], sem1).wait()
            pltpu.make_async_copy(label_hbm.at[idx_v.at[r]],
                                  lm_v.at[pl.ds(r * 128, 128)], sem2).wait()
        pltpu.sync_copy(fs_v, fs_hbm.at[pl.ds(base, GROWS)])
        pltpu.sync_copy(lm_v, lm_hbm.at[pl.ds(base, GROWS)])

    return body(feat, label, idx)


_MLP_R = 1024


def _mlp_body(fs_ref, yi_ref, w1f_ref, w1l_ref, b1_ref, w2_ref, b2_ref,
              w3_ref, b3_ref, wc_ref, bc_ref, alpha_ref, yo_ref, ye_ref):
    x = fs_ref[...]
    yi = yi_ref[...]
    h = (jnp.dot(x, w1f_ref[...], preferred_element_type=jnp.float32)
         + jnp.dot(yi, w1l_ref[...], preferred_element_type=jnp.float32)
         + b1_ref[...])
    h = jnp.maximum(h, 0.0)
    h = jnp.maximum(jnp.dot(h, w2_ref[...], preferred_element_type=jnp.float32)
                    + b2_ref[...], 0.0)
    h = h + jnp.maximum(jnp.dot(h, w3_ref[...], preferred_element_type=jnp.float32)
                        + b3_ref[...], 0.0)
    yo = jnp.dot(h, wc_ref[...], preferred_element_type=jnp.float32) + bc_ref[...]
    alpha = alpha_ref[0, 0]
    yo_ref[...] = yo
    ye_ref[...] = alpha * yi + (1.0 - alpha) * yo


def _mlp(fs, yi, W1, b1, W2, b2, W3, b3, Wc, bc, alpha):
    w1f = W1[:D_FEAT]
    w1l = W1[D_FEAT:]
    return pl.pallas_call(
        _mlp_body,
        grid=(K // _MLP_R,),
        in_specs=[
            pl.BlockSpec((_MLP_R, D_FEAT), lambda i: (i, 0)),
            pl.BlockSpec((_MLP_R, C), lambda i: (i, 0)),
            pl.BlockSpec((D_FEAT, H), lambda i: (0, 0)),
            pl.BlockSpec((C, H), lambda i: (0, 0)),
            pl.BlockSpec((1, H), lambda i: (0, 0)),
            pl.BlockSpec((H, H), lambda i: (0, 0)),
            pl.BlockSpec((1, H), lambda i: (0, 0)),
            pl.BlockSpec((H, H), lambda i: (0, 0)),
            pl.BlockSpec((1, H), lambda i: (0, 0)),
            pl.BlockSpec((H, C), lambda i: (0, 0)),
            pl.BlockSpec((1, C), lambda i: (0, 0)),
            pl.BlockSpec(memory_space=pltpu.SMEM),
        ],
        out_specs=[
            pl.BlockSpec((_MLP_R, C), lambda i: (i, 0)),
            pl.BlockSpec((_MLP_R, C), lambda i: (i, 0)),
        ],
        out_shape=[
            jax.ShapeDtypeStruct((K, C), jnp.float32),
            jax.ShapeDtypeStruct((K, C), jnp.float32),
        ],
    )(fs, yi, w1f, w1l, b1.reshape(1, H), W2, b2.reshape(1, H), W3,
      b3.reshape(1, H), Wc, bc.reshape(1, C), alpha.reshape(1, 1))


def kernel(feat, logits, label, W1, b1, W2, b2, W3, b3, Wc, bc, g):
    keys2d = _scores(logits)
    slots, vals = _slots(keys2d)
    idx_padded = _compact(slots, vals)
    idx = idx_padded[:K]

    fs, label_mask = _gather(feat, label, idx.reshape(K // 128, 128))
    yi = jnp.take(logits, idx, axis=0)
    alpha = jax.nn.sigmoid(g)
    yo, ye = _mlp(fs, yi, W1, b1, W2, b2, W3, b3, Wc, bc, alpha)

    yi_full = logits
    yo_full = logits.at[idx].set(yo, unique_indices=True)
    ye_full = logits.at[idx].set(ye, unique_indices=True)
    return (yi, yo, ye, yi_full, yo_full, ye_full, label, label_mask)


# speedup vs baseline: 3.3415x; 1.6502x over previous
import functools

import jax
import jax.numpy as jnp
from jax import lax
from jax.experimental import pallas as pl
from jax.experimental.pallas import tpu as pltpu
from jax.experimental.pallas import tpu_sc as plsc

N = 65536
D_FEAT = 128
C = 20
K = 8192
H = 256

NSUB = 32
GROWS = K // NSUB
SROWS = (N // 128) // NSUB
PAD = N - K

_SC_MESH = functools.partial(
    plsc.VectorSubcoreMesh, core_axis_name="c", subcore_axis_name="s")


_SCORE_R = 4096


def _score_body(l_ref, keys_ref):
    l = l_ref[...]
    lmax = jnp.max(l, axis=-1, keepdims=True)
    e = jnp.exp(l - lmax)
    p = e / jnp.sum(e, axis=-1, keepdims=True)
    s = -jnp.sum(p * jnp.log(p + 1e-9), axis=-1)
    u = lax.bitcast_convert_type(s, jnp.uint32)
    mask = jnp.where(u >> 31 == 1, jnp.uint32(0xFFFFFFFF), jnp.uint32(0x80000000))
    keys_ref[...] = (u ^ mask).reshape(_SCORE_R // 128, 128)


def _scores(logits):
    return pl.pallas_call(
        _score_body,
        grid=(N // _SCORE_R,),
        in_specs=[pl.BlockSpec((_SCORE_R, C), lambda i: (i, 0))],
        out_specs=pl.BlockSpec((_SCORE_R // 128, 128), lambda i: (i, 0)),
        out_shape=jax.ShapeDtypeStruct((N // 128, 128), jnp.uint32),
    )(logits)


def _slots_body(keys_ref, slots_ref, vals_ref):
    keys = keys_ref[...]

    def step(i, prefix):
        cand = prefix | (jnp.uint32(1) << jnp.uint32(31 - i))
        cnt = jnp.sum((keys >= cand).astype(jnp.int32))
        return jnp.where(cnt >= K, cand, prefix)

    t = lax.fori_loop(0, 32, step, jnp.uint32(0))
    cnt_gt = jnp.sum((keys > t).astype(jnp.int32))
    m = (K - cnt_gt).astype(jnp.float32)

    gt = keys > t
    eq = keys == t
    eq_f = eq.astype(jnp.float32)

    nrow = keys.shape[0]
    tri_l = (lax.broadcasted_iota(jnp.int32, (128, 128), 0)
             <= lax.broadcasted_iota(jnp.int32, (128, 128), 1)).astype(jnp.float32)
    tri_r = (lax.broadcasted_iota(jnp.int32, (nrow, nrow), 1)
             < lax.broadcasted_iota(jnp.int32, (nrow, nrow), 0)).astype(jnp.float32)

    ceq = jnp.dot(eq_f, tri_l, preferred_element_type=jnp.float32)
    eq_off = jnp.dot(tri_r, ceq[:, 127:128], preferred_element_type=jnp.float32)
    tie_rank = ceq + eq_off
    sel = gt | (eq & (tie_rank <= m))

    sel_f = sel.astype(jnp.float32)
    csel = jnp.dot(sel_f, tri_l, preferred_element_type=jnp.float32)
    sel_off = jnp.dot(tri_r, csel[:, 127:128], preferred_element_type=jnp.float32)
    pos = (csel + sel_off).astype(jnp.int32) - 1

    flat = (128 * lax.broadcasted_iota(jnp.int32, keys.shape, 0)
            + lax.broadcasted_iota(jnp.int32, keys.shape, 1))
    slots_ref[...] = jnp.where(sel, pos, K + flat - (pos + 1))
    vals_ref[...] = flat


def _slots(keys):
    return pl.pallas_call(
        _slots_body,
        in_specs=[pl.BlockSpec(memory_space=pltpu.VMEM)],
        out_specs=[
            pl.BlockSpec(memory_space=pltpu.VMEM),
            pl.BlockSpec(memory_space=pltpu.VMEM),
        ],
        out_shape=[
            jax.ShapeDtypeStruct((N // 128, 128), jnp.int32),
            jax.ShapeDtypeStruct((N // 128, 128), jnp.int32),
        ],
    )(keys)


def _compact(slots, vals):
    TROWS = (N // 128) // 16

    @functools.partial(
        pl.kernel,
        mesh=_SC_MESH(),
        out_type=jax.ShapeDtypeStruct((K,), jnp.int32),
        scratch_types=[
            pltpu.VMEM((TROWS, 128), jnp.int32),
            pltpu.VMEM((TROWS, 128), jnp.int32),
            pltpu.VMEM((K // NSUB,), jnp.int32),
            pltpu.VMEM_SHARED((N,), jnp.int32),
            pltpu.SemaphoreType.DMA,
        ],
    )
    def body(slots_hbm, vals_hbm, out_hbm, slot_v, val_v, bounce_v, perm_sh, sem):
        cid = lax.axis_index("c")
        sid = lax.axis_index("s")
        wid = sid * 2 + cid
        pltpu.sync_copy(slots_hbm.at[pl.ds(sid * TROWS, TROWS)], slot_v)
        pltpu.sync_copy(vals_hbm.at[pl.ds(sid * TROWS, TROWS)], val_v)
        for r in range(TROWS):
            pltpu.async_copy(val_v.at[r], perm_sh.at[slot_v.at[r]], sem)
        for r in range(TROWS):
            pltpu.make_async_copy(val_v.at[r], perm_sh.at[slot_v.at[r]], sem).wait()
        plsc.subcore_barrier()
        base = wid * (K // NSUB)
        pltpu.sync_copy(perm_sh.at[pl.ds(base, K // NSUB)], bounce_v)
        pltpu.sync_copy(bounce_v, out_hbm.at[pl.ds(base, K // NSUB)])

    return body(slots, vals)


def _gather(feat, label, idx):
    @functools.partial(
        pl.kernel,
        mesh=_SC_MESH(),
        out_type=[
            jax.ShapeDtypeStruct((K, D_FEAT), jnp.float32),
            jax.ShapeDtypeStruct((K,), jnp.int32),
        ],
        scratch_types=[
            pltpu.VMEM((GROWS // 128, 128), jnp.int32),
            pltpu.VMEM((GROWS, D_FEAT), jnp.float32),
            pltpu.VMEM((GROWS,), jnp.int32),
            pltpu.SemaphoreType.DMA,
            pltpu.SemaphoreType.DMA,
        ],
    )
    def body(feat_hbm, label_hbm, idx_hbm,
             fs_hbm, lm_hbm, idx_v, fs_v, lm_v, sem1, sem2):
        wid = lax.axis_index("s") * 2 + lax.axis_index("c")
        base = wid * GROWS
        pltpu.sync_copy(idx_hbm.at[pl.ds(wid * (GROWS // 128), GROWS // 128)], idx_v)
        for r in range(GROWS // 128):
            pltpu.async_copy(feat_hbm.at[idx_v.at[r]],
                             fs_v.at[pl.ds(r * 128, 128)], sem1)
            pltpu.async_copy(label_hbm.at[idx_v.at[r]],
                             lm_v.at[pl.ds(r * 128, 128)], sem2)
        for r in range(GROWS // 128):
            pltpu.make_async_copy(feat_hbm.at[idx_v.at[r]],
                                  fs_v.at[pl.ds(r * 128, 128)], sem1).wait()
            pltpu.make_async_copy(label_hbm.at[idx_v.at[r]],
                                  lm_v.at[pl.ds(r * 128, 128)], sem2).wait()
        pltpu.sync_copy(fs_v, fs_hbm.at[pl.ds(base, GROWS)])
        pltpu.sync_copy(lm_v, lm_hbm.at[pl.ds(base, GROWS)])

    return body(feat, label, idx)


_MLP_R = 1024


def _mlp_body(fs_ref, yi_ref, w1f_ref, w1l_ref, b1_ref, w2_ref, b2_ref,
              w3_ref, b3_ref, wc_ref, bc_ref, alpha_ref, yo_ref, ye_ref):
    x = fs_ref[...]
    yi = yi_ref[...]
    h = (jnp.dot(x, w1f_ref[...], preferred_element_type=jnp.float32)
         + jnp.dot(yi, w1l_ref[...], preferred_element_type=jnp.float32)
         + b1_ref[...])
    h = jnp.maximum(h, 0.0)
    h = jnp.maximum(jnp.dot(h, w2_ref[...], preferred_element_type=jnp.float32)
                    + b2_ref[...], 0.0)
    h = h + jnp.maximum(jnp.dot(h, w3_ref[...], preferred_element_type=jnp.float32)
                        + b3_ref[...], 0.0)
    yo = jnp.dot(h, wc_ref[...], preferred_element_type=jnp.float32) + bc_ref[...]
    alpha = alpha_ref[0, 0]
    yo_ref[...] = yo
    ye_ref[...] = alpha * yi + (1.0 - alpha) * yo


def _mlp(fs, yi, W1, b1, W2, b2, W3, b3, Wc, bc, alpha):
    w1f = W1[:D_FEAT]
    w1l = W1[D_FEAT:]
    return pl.pallas_call(
        _mlp_body,
        grid=(K // _MLP_R,),
        in_specs=[
            pl.BlockSpec((_MLP_R, D_FEAT), lambda i: (i, 0)),
            pl.BlockSpec((_MLP_R, C), lambda i: (i, 0)),
            pl.BlockSpec((D_FEAT, H), lambda i: (0, 0)),
            pl.BlockSpec((C, H), lambda i: (0, 0)),
            pl.BlockSpec((1, H), lambda i: (0, 0)),
            pl.BlockSpec((H, H), lambda i: (0, 0)),
            pl.BlockSpec((1, H), lambda i: (0, 0)),
            pl.BlockSpec((H, H), lambda i: (0, 0)),
            pl.BlockSpec((1, H), lambda i: (0, 0)),
            pl.BlockSpec((H, C), lambda i: (0, 0)),
            pl.BlockSpec((1, C), lambda i: (0, 0)),
            pl.BlockSpec(memory_space=pltpu.SMEM),
        ],
        out_specs=[
            pl.BlockSpec((_MLP_R, C), lambda i: (i, 0)),
            pl.BlockSpec((_MLP_R, C), lambda i: (i, 0)),
        ],
        out_shape=[
            jax.ShapeDtypeStruct((K, C), jnp.float32),
            jax.ShapeDtypeStruct((K, C), jnp.float32),
        ],
    )(fs, yi, w1f, w1l, b1.reshape(1, H), W2, b2.reshape(1, H), W3,
      b3.reshape(1, H), Wc, bc.reshape(1, C), alpha.reshape(1, 1))


def kernel(feat, logits, label, W1, b1, W2, b2, W3, b3, Wc, bc, g):
    keys2d = _scores(logits)
    slots, vals = _slots(keys2d)
    idx = _compact(slots, vals)

    fs, label_mask = _gather(feat, label, idx.reshape(K // 128, 128))
    yi = jnp.take(logits, idx, axis=0)
    alpha = jax.nn.sigmoid(g)
    yo, ye = _mlp(fs, yi, W1, b1, W2, b2, W3, b3, Wc, bc, alpha)

    yi_full = logits
    yo_full = logits.at[idx].set(yo, unique_indices=True)
    ye_full = logits.at[idx].set(ye, unique_indices=True)
    return (yi, yo, ye, yi_full, yo_full, ye_full, label, label_mask)
